# baseline (speedup 1.0000x reference)
import functools

import jax
import jax.numpy as jnp
from jax import lax
from jax.experimental import pallas as pl
from jax.experimental.pallas import tpu as pltpu
from jax.experimental.pallas import tpu_sc as plsc

_NMONTH, _NTID, _NWEEK, _NHOL, _NDT = 12, 288, 7, 2, 3
_NCOMBO = _NMONTH * _NWEEK * _NHOL * _NDT
_HID = 128
_CBLK = 8

_NC, _NS = 2, 16
_NW = _NC * _NS
_CHUNK = 128


def _build_table_body(month_ref, tid_ref, week_ref, hol_ref, dt_ref, w_ref,
                      b_ref, out_ref):
    i = pl.program_id(0)
    c = i * _CBLK + lax.broadcasted_iota(jnp.int32, (_CBLK, 1), 0)
    m = c // (_NWEEK * _NHOL * _NDT)
    w = (c // (_NHOL * _NDT)) % _NWEEK
    h = (c // _NDT) % _NHOL
    d = c % _NDT

    def onehot(idx, n):
        return (idx == lax.broadcasted_iota(jnp.int32, (_CBLK, n), 1)
                ).astype(jnp.float32)

    wmat = w_ref[...]
    proj_m = jnp.dot(month_ref[...], wmat[0:4, :], preferred_element_type=jnp.float32)
    proj_w = jnp.dot(week_ref[...], wmat[8:12, :], preferred_element_type=jnp.float32)
    proj_h = jnp.dot(hol_ref[...], wmat[12:14, :], preferred_element_type=jnp.float32)
    proj_d = jnp.dot(dt_ref[...], wmat[14:16, :], preferred_element_type=jnp.float32)
    combo = (jnp.dot(onehot(m, _NMONTH), proj_m, preferred_element_type=jnp.float32)
             + jnp.dot(onehot(w, _NWEEK), proj_w, preferred_element_type=jnp.float32)
             + jnp.dot(onehot(h, _NHOL), proj_h, preferred_element_type=jnp.float32)
             + jnp.dot(onehot(d, _NDT), proj_d, preferred_element_type=jnp.float32)
             + b_ref[...])
    tid_proj = jnp.dot(tid_ref[...], wmat[4:8, :],
                       preferred_element_type=jnp.float32)
    out_ref[...] = combo[:, None, :] + tid_proj[None, :, :]


def _build_table(month_tab, tid_tab, week_tab, holiday_tab, date_type_tab,
                 fuse_W, fuse_b):
    full = lambda s: pl.BlockSpec(s, lambda i: (0,) * len(s))
    return pl.pallas_call(
        _build_table_body,
        grid=(_NCOMBO // _CBLK,),
        in_specs=[
            full((_NMONTH, 4)), full((_NTID, 4)), full((_NWEEK, 4)),
            full((_NHOL, 2)), full((_NDT, 2)), full((16, _HID)),
            full((1, _HID)),
        ],
        out_specs=pl.BlockSpec((_CBLK, _NTID, _HID), lambda i: (i, 0, 0)),
        out_shape=jax.ShapeDtypeStruct((_NCOMBO, _NTID, _HID), jnp.float32),
    )(month_tab, tid_tab, week_tab, holiday_tab, date_type_tab, fuse_W,
      fuse_b.reshape(1, _HID))


def _make_sc_gather(n_tokens):
    n_per_w = n_tokens // _NW
    n_chunks = n_per_w // _CHUNK
    mesh = plsc.VectorSubcoreMesh(core_axis_name="c", subcore_axis_name="s")

    @functools.partial(
        pl.kernel,
        mesh=mesh,
        out_type=jax.ShapeDtypeStruct((n_tokens, _HID), jnp.float32),
        scratch_types=[
            pltpu.VMEM((5 * _CHUNK,), jnp.int32),
            pltpu.VMEM((_CHUNK,), jnp.int32),
            pltpu.VMEM((_CHUNK, _HID), jnp.float32),
            pltpu.SemaphoreType.DMA,
        ],
        compiler_params=pltpu.CompilerParams(needs_layout_passes=False),
    )
    def sc_gather(tf_hbm, p_hbm, out_hbm, tf_v, idx_v, rows_v, sem):
        wid = lax.axis_index("s") * _NC + lax.axis_index("c")
        tile_base = wid * n_per_w

        def chunk(k, carry):
            base = tile_base + k * _CHUNK
            pltpu.sync_copy(tf_hbm.at[pl.ds(base * 5, _CHUNK * 5)], tf_v)

            def ivec(i, carry2):
                lane5 = (i * 16 + lax.iota(jnp.int32, 16)) * 5
                m = plsc.load_gather(tf_v, [lane5])
                t = plsc.load_gather(tf_v, [lane5 + 1])
                w = plsc.load_gather(tf_v, [lane5 + 2])
                h = plsc.load_gather(tf_v, [lane5 + 3])
                d = plsc.load_gather(tf_v, [lane5 + 4])
                row = ((((m - 1) * _NWEEK + w) * _NHOL + h) * _NDT + d) * _NTID + t
                idx_v[pl.ds(i * 16, 16)] = row
                return carry2

            lax.fori_loop(0, _CHUNK // 16, ivec, 0, unroll=True)
            pltpu.async_copy(p_hbm.at[idx_v], rows_v, sem).wait()
            pltpu.sync_copy(rows_v, out_hbm.at[pl.ds(base, _CHUNK)])
            return carry

        lax.fori_loop(0, n_chunks, chunk, 0)

    return sc_gather


def kernel(time_features, month_tab, tid_tab, week_tab, holiday_tab,
           date_type_tab, fuse_W, fuse_b):
    b, l, _ = time_features.shape
    n_tokens = b * l
    table = _build_table(month_tab, tid_tab, week_tab, holiday_tab,
                         date_type_tab, fuse_W, fuse_b)
    flat = _make_sc_gather(n_tokens)(
        time_features.reshape(-1), table.reshape(_NCOMBO * _NTID, _HID))
    return flat.reshape(b, l, _HID)


# speedup vs baseline: 1.0006x; 1.0006x over previous
import functools

import jax
import jax.numpy as jnp
from jax import lax
from jax.experimental import pallas as pl
from jax.experimental.pallas import tpu as pltpu
from jax.experimental.pallas import tpu_sc as plsc

_NMONTH, _NTID, _NWEEK, _NHOL, _NDT = 12, 288, 7, 2, 3
_NCOMBO = _NMONTH * _NWEEK * _NHOL * _NDT
_HID = 128
_CBLK = 8

_NC, _NS = 2, 16
_NW = _NC * _NS
_CHUNK = 128


def _build_table_body(month_ref, tid_ref, week_ref, hol_ref, dt_ref, w_ref,
                      b_ref, out_ref):
    i = pl.program_id(0)
    c = i * _CBLK + lax.broadcasted_iota(jnp.int32, (_CBLK, 1), 0)
    m = c // (_NWEEK * _NHOL * _NDT)
    w = (c // (_NHOL * _NDT)) % _NWEEK
    h = (c // _NDT) % _NHOL
    d = c % _NDT

    def onehot(idx, n):
        return (idx == lax.broadcasted_iota(jnp.int32, (_CBLK, n), 1)
                ).astype(jnp.float32)

    wmat = w_ref[...]
    proj_m = jnp.dot(month_ref[...], wmat[0:4, :], preferred_element_type=jnp.float32)
    proj_w = jnp.dot(week_ref[...], wmat[8:12, :], preferred_element_type=jnp.float32)
    proj_h = jnp.dot(hol_ref[...], wmat[12:14, :], preferred_element_type=jnp.float32)
    proj_d = jnp.dot(dt_ref[...], wmat[14:16, :], preferred_element_type=jnp.float32)
    combo = (jnp.dot(onehot(m, _NMONTH), proj_m, preferred_element_type=jnp.float32)
             + jnp.dot(onehot(w, _NWEEK), proj_w, preferred_element_type=jnp.float32)
             + jnp.dot(onehot(h, _NHOL), proj_h, preferred_element_type=jnp.float32)
             + jnp.dot(onehot(d, _NDT), proj_d, preferred_element_type=jnp.float32)
             + b_ref[...])
    tid_proj = jnp.dot(tid_ref[...], wmat[4:8, :],
                       preferred_element_type=jnp.float32)
    out_ref[...] = combo[:, None, :] + tid_proj[None, :, :]


def _build_table(month_tab, tid_tab, week_tab, holiday_tab, date_type_tab,
                 fuse_W, fuse_b):
    full = lambda s: pl.BlockSpec(s, lambda i: (0,) * len(s))
    return pl.pallas_call(
        _build_table_body,
        grid=(_NCOMBO // _CBLK,),
        in_specs=[
            full((_NMONTH, 4)), full((_NTID, 4)), full((_NWEEK, 4)),
            full((_NHOL, 2)), full((_NDT, 2)), full((16, _HID)),
            full((1, _HID)),
        ],
        out_specs=pl.BlockSpec((_CBLK, _NTID, _HID), lambda i: (i, 0, 0)),
        out_shape=jax.ShapeDtypeStruct((_NCOMBO, _NTID, _HID), jnp.float32),
    )(month_tab, tid_tab, week_tab, holiday_tab, date_type_tab, fuse_W,
      fuse_b.reshape(1, _HID))


_NBUF = 5
_STAGE = 2560


def _make_sc_gather(n_tokens):
    n_per_w = n_tokens // _NW
    n_chunks = n_per_w // _CHUNK
    n_rounds = n_chunks // _NBUF
    n_stage = n_per_w // _STAGE
    mesh = plsc.VectorSubcoreMesh(core_axis_name="c", subcore_axis_name="s")

    @functools.partial(
        pl.kernel,
        mesh=mesh,
        out_type=jax.ShapeDtypeStruct((n_tokens, _HID), jnp.float32),
        scratch_types=[
            pltpu.VMEM((5 * _STAGE,), jnp.int32),
            pltpu.VMEM((n_per_w,), jnp.int32),
            [pltpu.VMEM((_CHUNK, _HID), jnp.float32) for _ in range(_NBUF)],
            pltpu.SemaphoreType.DMA((_NBUF,)),
            pltpu.SemaphoreType.DMA((_NBUF,)),
        ],
        compiler_params=pltpu.CompilerParams(needs_layout_passes=False),
    )
    def sc_gather(tf_hbm, p_hbm, out_hbm, tf_v, idx_v, rows, gsem, ssem):
        wid = lax.axis_index("s") * _NC + lax.axis_index("c")
        tile_base = wid * n_per_w

        def stage(j, carry):
            pltpu.sync_copy(
                tf_hbm.at[pl.ds((tile_base + j * _STAGE) * 5, 5 * _STAGE)], tf_v)

            def ivec(i, carry2):
                lane5 = (i * 16 + lax.iota(jnp.int32, 16)) * 5
                m = plsc.load_gather(tf_v, [lane5])
                t = plsc.load_gather(tf_v, [lane5 + 1])
                w = plsc.load_gather(tf_v, [lane5 + 2])
                h = plsc.load_gather(tf_v, [lane5 + 3])
                d = plsc.load_gather(tf_v, [lane5 + 4])
                row = ((((m - 1) * _NWEEK + w) * _NHOL + h) * _NDT + d) * _NTID + t
                idx_v[pl.ds(j * _STAGE + i * 16, 16)] = row
                return carry2

            lax.fori_loop(0, _STAGE // 16, ivec, 0)
            return carry

        lax.fori_loop(0, n_stage, stage, 0)

        def fire_gather(g, b):
            pltpu.async_copy(
                p_hbm.at[idx_v.at[pl.ds(g * _CHUNK, _CHUNK)]], rows[b],
                gsem.at[b])

        def fire_scatter(g, b):
            pltpu.async_copy(
                rows[b], out_hbm.at[pl.ds(tile_base + g * _CHUNK, _CHUNK)],
                ssem.at[b])

        for b in range(_NBUF):
            fire_gather(b, b)

        def round_body(r, carry):
            for b in range(_NBUF):
                pltpu.make_async_copy(
                    p_hbm.at[idx_v.at[pl.ds(0, _CHUNK)]], rows[b],
                    gsem.at[b]).wait()
                fire_scatter((r - 1) * _NBUF + b, b)
            for b in range(_NBUF):
                pltpu.make_async_copy(
                    rows[b], out_hbm.at[pl.ds(tile_base, _CHUNK)],
                    ssem.at[b]).wait()
                fire_gather(r * _NBUF + b, b)
            return carry

        lax.fori_loop(1, n_rounds, round_body, 0)

        for b in range(_NBUF):
            pltpu.make_async_copy(
                p_hbm.at[idx_v.at[pl.ds(0, _CHUNK)]], rows[b], gsem.at[b]).wait()
            fire_scatter((n_rounds - 1) * _NBUF + b, b)
        for b in range(_NBUF):
            pltpu.make_async_copy(
                rows[b], out_hbm.at[pl.ds(tile_base, _CHUNK)], ssem.at[b]).wait()

    return sc_gather


def kernel(time_features, month_tab, tid_tab, week_tab, holiday_tab,
           date_type_tab, fuse_W, fuse_b):
    b, l, _ = time_features.shape
    n_tokens = b * l
    table = _build_table(month_tab, tid_tab, week_tab, holiday_tab,
                         date_type_tab, fuse_W, fuse_b)
    flat = _make_sc_gather(n_tokens)(
        time_features.reshape(-1), table.reshape(_NCOMBO * _NTID, _HID))
    return flat.reshape(b, l, _HID)


# speedup vs baseline: 10.8044x; 10.7979x over previous
import functools

import jax
import jax.numpy as jnp
from jax import lax
from jax.experimental import pallas as pl
from jax.experimental.pallas import tpu as pltpu
from jax.experimental.pallas import tpu_sc as plsc

_NMONTH, _NTID, _NWEEK, _NHOL, _NDT = 12, 288, 7, 2, 3
_NCOMBO = _NMONTH * _NWEEK * _NHOL * _NDT
_NROWS = 896
_HID = 128

_NC, _NS = 2, 16
_NW = _NC * _NS
_CHUNK = 128
_NREP = 32


def _build_table_body(month_ref, tid_ref, week_ref, hol_ref, dt_ref, w_ref,
                      b_ref, out_ref):
    c = lax.broadcasted_iota(jnp.int32, (_NCOMBO, 1), 0)
    m = c // (_NWEEK * _NHOL * _NDT)
    w = (c // (_NHOL * _NDT)) % _NWEEK
    h = (c // _NDT) % _NHOL
    d = c % _NDT

    def onehot(idx, n):
        return (idx == lax.broadcasted_iota(jnp.int32, (_NCOMBO, n), 1)
                ).astype(jnp.float32)

    wmat = w_ref[...]
    proj_m = jnp.dot(month_ref[...], wmat[0:4, :], preferred_element_type=jnp.float32)
    proj_w = jnp.dot(week_ref[...], wmat[8:12, :], preferred_element_type=jnp.float32)
    proj_h = jnp.dot(hol_ref[...], wmat[12:14, :], preferred_element_type=jnp.float32)
    proj_d = jnp.dot(dt_ref[...], wmat[14:16, :], preferred_element_type=jnp.float32)
    combo = (jnp.dot(onehot(m, _NMONTH), proj_m, preferred_element_type=jnp.float32)
             + jnp.dot(onehot(w, _NWEEK), proj_w, preferred_element_type=jnp.float32)
             + jnp.dot(onehot(h, _NHOL), proj_h, preferred_element_type=jnp.float32)
             + jnp.dot(onehot(d, _NDT), proj_d, preferred_element_type=jnp.float32)
             + b_ref[...])
    tid_proj = jnp.dot(tid_ref[...], wmat[4:8, :],
                       preferred_element_type=jnp.float32)
    out_ref[0:_NCOMBO, :] = combo
    out_ref[_NCOMBO:_NCOMBO + _NTID, :] = tid_proj
    out_ref[_NCOMBO + _NTID:_NROWS, :] = jnp.zeros(
        (_NROWS - _NCOMBO - _NTID, _HID), jnp.float32)


def _build_table(month_tab, tid_tab, week_tab, holiday_tab, date_type_tab,
                 fuse_W, fuse_b):
    full = lambda s: pl.BlockSpec(s, lambda i: (0,) * len(s))
    return pl.pallas_call(
        _build_table_body,
        grid=(_NREP,),
        in_specs=[
            full((_NMONTH, 4)), full((_NTID, 4)), full((_NWEEK, 4)),
            full((_NHOL, 2)), full((_NDT, 2)), full((16, _HID)),
            full((1, _HID)),
        ],
        out_specs=pl.BlockSpec((_NROWS, _HID), lambda i: (i, 0)),
        out_shape=jax.ShapeDtypeStruct((_NREP * _NROWS, _HID), jnp.float32),
    )(month_tab, tid_tab, week_tab, holiday_tab, date_type_tab, fuse_W,
      fuse_b.reshape(1, _HID))


def _make_sc_gather(n_tokens):
    n_per_w = n_tokens // _NW
    n_chunks = n_per_w // _CHUNK
    mesh = plsc.VectorSubcoreMesh(core_axis_name="c", subcore_axis_name="s")

    @functools.partial(
        pl.kernel,
        mesh=mesh,
        out_type=jax.ShapeDtypeStruct((n_tokens, _HID), jnp.float32),
        scratch_types=[
            [pltpu.VMEM((5 * _CHUNK,), jnp.int32) for _ in range(2)],
            [pltpu.VMEM((_CHUNK, _HID), jnp.float32) for _ in range(2)],
            pltpu.VMEM((_CHUNK, _HID), jnp.float32),
            pltpu.VMEM((_CHUNK,), jnp.int32),
            pltpu.VMEM((_CHUNK,), jnp.int32),
            pltpu.SemaphoreType.DMA((2,)),
            pltpu.SemaphoreType.DMA((2,)),
            pltpu.SemaphoreType.DMA,
            pltpu.SemaphoreType.DMA,
        ],
        compiler_params=pltpu.CompilerParams(needs_layout_passes=False),
    )
    def sc_gather(tf_hbm, tab_hbm, out_hbm, tf_v, buf_a, buf_b,
                  cidx, tidx, tfsem, ssem, gsema, gsemb):
        sid = lax.axis_index("s")
        wid = sid * _NC + lax.axis_index("c")
        tile_base = wid * n_per_w

        def tf_slice(g):
            return tf_hbm.at[pl.ds((tile_base + g * _CHUNK) * 5, 5 * _CHUNK)]

        def chunk_body(g, b, first):
            pltpu.sync_copy(tf_slice(g), tf_v[b])
            for i in range(_CHUNK // 16):
                pos = i * 16 + lax.iota(jnp.int32, 16)
                lane5 = pos * 5
                m = plsc.load_gather(tf_v[b], [lane5])
                t = plsc.load_gather(tf_v[b], [lane5 + 1])
                w = plsc.load_gather(tf_v[b], [lane5 + 2])
                h = plsc.load_gather(tf_v[b], [lane5 + 3])
                d = plsc.load_gather(tf_v[b], [lane5 + 4])
                rep = ((pos + wid) % _NREP) * _NROWS
                cidx[pl.ds(i * 16, 16)] = (
                    rep + (((m - 1) * _NWEEK + w) * _NHOL + h) * _NDT + d)
                tidx[pl.ds(i * 16, 16)] = rep + t + _NCOMBO
            if not first:
                pltpu.make_async_copy(
                    buf_a[b], out_hbm.at[pl.ds(tile_base, _CHUNK)],
                    ssem.at[b]).wait()
            pltpu.async_copy(tab_hbm.at[cidx], buf_a[b], gsema)
            pltpu.async_copy(tab_hbm.at[tidx], buf_b, gsemb)
            pltpu.make_async_copy(tab_hbm.at[cidx], buf_a[b], gsema).wait()
            pltpu.make_async_copy(tab_hbm.at[tidx], buf_b, gsemb).wait()

            def addj(j, carry):
                for kk in range(_HID // 16):
                    buf_a[b][j, pl.ds(kk * 16, 16)] = (
                        buf_a[b][j, pl.ds(kk * 16, 16)]
                        + buf_b[j, pl.ds(kk * 16, 16)])
                return carry

            lax.fori_loop(0, _CHUNK, addj, 0)
            pltpu.async_copy(
                buf_a[b], out_hbm.at[pl.ds(tile_base + g * _CHUNK, _CHUNK)],
                ssem.at[b])

        chunk_body(0, 0, True)
        chunk_body(1, 1, True)

        def round_body(r, carry):
            chunk_body(2 * r, 0, False)
            chunk_body(2 * r + 1, 1, False)
            return carry

        lax.fori_loop(1, n_chunks // 2, round_body, 0)
        for b in range(2):
            pltpu.make_async_copy(
                buf_a[b], out_hbm.at[pl.ds(tile_base, _CHUNK)], ssem.at[b]).wait()

    return sc_gather


def kernel(time_features, month_tab, tid_tab, week_tab, holiday_tab,
           date_type_tab, fuse_W, fuse_b):
    b, l, _ = time_features.shape
    n_tokens = b * l
    table = _build_table(month_tab, tid_tab, week_tab, holiday_tab,
                         date_type_tab, fuse_W, fuse_b)
    flat = _make_sc_gather(n_tokens)(time_features.reshape(-1), table)
    return flat.reshape(b, l, _HID)
